# baseline (speedup 1.0000x reference)
import functools

import jax
import jax.numpy as jnp
from jax import lax
from jax.experimental import pallas as pl
from jax.experimental.pallas import tpu as pltpu
from jax.experimental.pallas import tpu_sc as plsc

B = 16384
S = 200
SP = 256
E = 32
NCLS = 10

NC = 2
NS = 16
NW = NC * NS
BPW = B // NW
HALF = BPW // 2
CH = 128
NCH = HALF // CH
NBUF = 2
RING = 4


def _tc_repack(text):
  BLK = 2048

  def body(x_ref, o_ref):
    x = x_ref[...]
    xp = jnp.concatenate(
        [x, jnp.zeros((BLK, SP - S), jnp.int32)], axis=1)
    o_ref[...] = xp.reshape(2 * BLK, 128)

  return pl.pallas_call(
      body,
      grid=(B // BLK,),
      in_specs=[pl.BlockSpec((BLK, S), lambda i: (i, 0))],
      out_specs=pl.BlockSpec((2 * BLK, 128), lambda i: (i, 0)),
      out_shape=jax.ShapeDtypeStruct((2 * B, 128), jnp.int32),
  )(text)


def _sc_embed_sum(table, text2):
  mesh = plsc.VectorSubcoreMesh(
      core_axis_name="c", subcore_axis_name="s", num_cores=NC,
      num_subcores=NS)

  @functools.partial(
      pl.kernel,
      out_type=jax.ShapeDtypeStruct((B, E), jnp.float32),
      mesh=mesh,
      scratch_types=[
          pltpu.VMEM((2 * HALF, 128), jnp.int32),
          pltpu.VMEM((RING, NCH, CH), jnp.int32),
          pltpu.VMEM((BPW, E), jnp.float32),
          pltpu.SemaphoreType.DMA,
          pltpu.SemaphoreType.DMA,
      ],
      compiler_params=pltpu.CompilerParams(
          use_tc_tiling_on_sc=False, needs_layout_passes=False),
  )
  def body(table_hbm, text_hbm, out_hbm, idx_nat, idx_ring, acc_v,
           sem_idx, sem_g):
    wid = lax.axis_index("s") * NC + lax.axis_index("c")
    base = wid * BPW

    ar = jnp.arange(16, dtype=jnp.int32)
    zeros = jnp.zeros((16,), jnp.float32)

    def zbody(i, carry):
      acc_v[i, pl.ds(0, 16)] = zeros
      acc_v[i, pl.ds(16, 16)] = zeros
      return carry

    lax.fori_loop(0, BPW, zbody, 0, unroll=4)

    for h in range(2):
      pltpu.async_copy(
          text_hbm.at[pl.ds((base + h * HALF) * 2, 2 * HALF)],
          idx_nat, sem_idx).wait()

      def transpose_step(s, slot):
        row_hi = lax.div(s, 128)
        col = jnp.zeros((16,), jnp.int32) + lax.rem(s, 128)
        for c in range(NCH):
          for k in range(CH // 16):
            rows = (ar + (c * CH + k * 16)) * 2 + row_hi
            idx_ring[slot, c, pl.ds(k * 16, 16)] = plsc.load_gather(
                idx_nat, [rows, col])

      def fire(slot, h=h):
        for c in range(NCH):
          pltpu.async_copy(
              table_hbm.at[idx_ring.at[slot, c]],
              acc_v.at[pl.ds(h * HALF + c * CH, CH)],
              sem_g, add=True)

      def drain_one():
        pltpu.make_async_copy(
            table_hbm.at[idx_ring.at[0, 0]],
            acc_v.at[pl.ds(0, CH)], sem_g).wait()

      for j in range(NBUF):
        transpose_step(jnp.int32(j), jnp.int32(j))
        fire(jnp.int32(j))

      def gbody(s, carry):
        slot = lax.rem(s, RING)
        transpose_step(s, slot)
        for _ in range(NCH):
          drain_one()
        fire(slot)
        return carry

      lax.fori_loop(NBUF, S, gbody, 0)
      for j in range(NBUF * NCH):
        drain_one()

    pltpu.async_copy(acc_v, out_hbm.at[pl.ds(base, BPW)], sem_idx).wait()

  return body(table, text2)


def _tc_head(sums, w_pad, b_pad):
  BLK = 2048

  def body(x_ref, w_ref, b_ref, o_ref):
    acc = jnp.dot(x_ref[...], w_ref[...], preferred_element_type=jnp.float32)
    o_ref[...] = acc * (1.0 / S) + b_ref[...]

  return pl.pallas_call(
      body,
      grid=(B // BLK,),
      in_specs=[
          pl.BlockSpec((BLK, E), lambda i: (i, 0)),
          pl.BlockSpec((E, 128), lambda i: (0, 0)),
          pl.BlockSpec((1, 128), lambda i: (0, 0)),
      ],
      out_specs=pl.BlockSpec((BLK, 128), lambda i: (i, 0)),
      out_shape=jax.ShapeDtypeStruct((B, 128), jnp.float32),
  )(sums, w_pad, b_pad)


@jax.jit
def kernel(text, table, W, b):
  text2 = _tc_repack(text)
  sums = _sc_embed_sum(table, text2)
  w_pad = jnp.pad(W, ((0, 0), (0, 128 - NCLS)))
  b_pad = jnp.pad(b, (0, 128 - NCLS)).reshape(1, 128)
  logits = _tc_head(sums, w_pad, b_pad)
  return logits[:, :NCLS]


# speedup vs baseline: 1.2301x; 1.2301x over previous
import functools

import jax
import jax.numpy as jnp
from jax import lax
from jax.experimental import pallas as pl
from jax.experimental.pallas import tpu as pltpu
from jax.experimental.pallas import tpu_sc as plsc

B = 16384
S = 200
SP = 256
E = 32
V = 1000000
NCLS = 10

NC = 2
NS = 16
NW = NC * NS
BPW = B // NW
CH = 128
NQ = BPW // CH
NBUF = 8
RING = 16


def _tc_repack_table(table_t):
  BLKV = 16384

  def body(x_ref, o_ref):
    x = x_ref[...]
    xt = jnp.transpose(x)
    x3 = xt.reshape(BLKV // 4, 4, E)
    o_ref[...] = jnp.concatenate(
        [x3[:, q, :] for q in range(4)], axis=1)

  return pl.pallas_call(
      body,
      grid=(pl.cdiv(V, BLKV),),
      in_specs=[pl.BlockSpec((E, BLKV), lambda i: (0, i))],
      out_specs=pl.BlockSpec((BLKV * E // 128, 128), lambda i: (i, 0)),
      out_shape=jax.ShapeDtypeStruct((V * E // 128, 128), jnp.float32),
  )(table_t)


def _tc_repack_text(text):
  BLK = 2048

  def body(x_ref, o_ref):
    x = x_ref[...]
    xp = jnp.concatenate(
        [x, jnp.zeros((BLK, SP - S), jnp.int32)], axis=1)
    o_ref[...] = xp.reshape(2 * BLK, 128)

  return pl.pallas_call(
      body,
      grid=(B // BLK,),
      in_specs=[pl.BlockSpec((BLK, S), lambda i: (i, 0))],
      out_specs=pl.BlockSpec((2 * BLK, 128), lambda i: (i, 0)),
      out_shape=jax.ShapeDtypeStruct((2 * B, 128), jnp.int32),
  )(text)


def _sc_embed_sum(table, text2):
  mesh = plsc.VectorSubcoreMesh(
      core_axis_name="c", subcore_axis_name="s", num_cores=NC,
      num_subcores=NS)

  @functools.partial(
      pl.kernel,
      out_type=jax.ShapeDtypeStruct((B, E), jnp.float32),
      mesh=mesh,
      scratch_types=[
          pltpu.VMEM((2, 2 * CH, 128), jnp.int32),
          pltpu.VMEM((RING, CH), jnp.int32),
          pltpu.VMEM((BPW, E), jnp.float32),
          pltpu.SemaphoreType.DMA,
          pltpu.SemaphoreType.DMA,
      ],
      compiler_params=pltpu.CompilerParams(
          use_tc_tiling_on_sc=False, needs_layout_passes=False),
  )
  def body(table_hbm, text_hbm, out_hbm, idx_nat, idx_ring, acc_v,
           sem_idx, sem_g):
    wid = lax.axis_index("s") * NC + lax.axis_index("c")
    base = wid * BPW

    ar = jnp.arange(16, dtype=jnp.int32)
    zeros = jnp.zeros((16,), jnp.float32)

    def zbody(i, carry):
      acc_v[i, pl.ds(0, 16)] = zeros
      acc_v[i, pl.ds(16, 16)] = zeros
      return carry

    lax.fori_loop(0, BPW, zbody, 0, unroll=4)

    def stage(q):
      return pltpu.async_copy(
          text_hbm.at[pl.ds((base + q * CH) * 2, 2 * CH)],
          idx_nat.at[q % 2], sem_idx)

    stage(0).wait()
    for q in range(NQ):
      if q + 1 < NQ:
        stage(q + 1)

      buf = idx_nat.at[q % 2]

      def transpose_step(s, slot, buf=buf):
        row_hi = lax.div(s, 128)
        col = jnp.zeros((16,), jnp.int32) + lax.rem(s, 128)
        for k in range(CH // 16):
          rows = (ar + k * 16) * 2 + row_hi
          idx_ring[slot, pl.ds(k * 16, 16)] = plsc.load_gather(
              buf, [rows, col])

      def fire(slot, q=q):
        pltpu.async_copy(
            table_hbm.at[idx_ring.at[slot]],
            acc_v.at[pl.ds(q * CH, CH)],
            sem_g, add=True)

      def drain_one():
        pltpu.make_async_copy(
            table_hbm.at[idx_ring.at[0]],
            acc_v.at[pl.ds(0, CH)], sem_g).wait()

      for j in range(NBUF):
        transpose_step(jnp.int32(j), jnp.int32(j))
        fire(jnp.int32(j))

      def gbody(s, carry):
        slot = lax.rem(s, RING)
        transpose_step(s, slot)
        drain_one()
        fire(slot)
        return carry

      lax.fori_loop(NBUF, S, gbody, 0)
      for j in range(NBUF):
        drain_one()

      if q + 1 < NQ:
        pltpu.make_async_copy(
            text_hbm.at[pl.ds(0, 2 * CH)], idx_nat.at[0], sem_idx).wait()

    pltpu.async_copy(acc_v, out_hbm.at[pl.ds(base, BPW)], sem_idx).wait()

  return body(table, text2)


def _tc_head(sums, w_pad, b_pad):
  BLK = 2048

  def body(x_ref, w_ref, b_ref, o_ref):
    acc = jnp.dot(x_ref[...], w_ref[...], preferred_element_type=jnp.float32)
    o_ref[...] = acc * (1.0 / S) + b_ref[...]

  return pl.pallas_call(
      body,
      grid=(B // BLK,),
      in_specs=[
          pl.BlockSpec((BLK, E), lambda i: (i, 0)),
          pl.BlockSpec((E, 128), lambda i: (0, 0)),
          pl.BlockSpec((1, 128), lambda i: (0, 0)),
      ],
      out_specs=pl.BlockSpec((BLK, 128), lambda i: (i, 0)),
      out_shape=jax.ShapeDtypeStruct((B, 128), jnp.float32),
  )(sums, w_pad, b_pad)


@jax.jit
def kernel(text, table, W, b):
  table_lin = _tc_repack_table(table.T)
  text2 = _tc_repack_text(text)
  sums = _sc_embed_sum(table_lin.reshape(V, E), text2)
  w_pad = jnp.pad(W, ((0, 0), (0, 128 - NCLS)))
  b_pad = jnp.pad(b, (0, 128 - NCLS)).reshape(1, 128)
  logits = _tc_head(sums, w_pad, b_pad)
  return logits[:, :NCLS]


# speedup vs baseline: 1.4857x; 1.2078x over previous
import functools

import jax
import jax.numpy as jnp
from jax import lax
from jax.experimental import pallas as pl
from jax.experimental.pallas import tpu as pltpu
from jax.experimental.pallas import tpu_sc as plsc

B = 16384
S = 200
SP = 256
E = 32
V = 1000000
NCLS = 10
NP = 16

NC = 2
NS = 16
NW = NC * NS
BPW = B // NW
CH = 128
NQ = BPW // CH
NBUF = 8
RING = 16


def _tc_repack_table(table_t, w2):
  BLKV = 16384

  def body(x_ref, w_ref, o_ref):
    x = x_ref[...]
    p = lax.dot_general(
        x, w_ref[...], (((0,), (0,)), ((), ())),
        preferred_element_type=jnp.float32)
    p3 = p.reshape(BLKV // 8, 8, NP)
    o_ref[...] = jnp.concatenate(
        [p3[:, q, :] for q in range(8)], axis=1)

  return pl.pallas_call(
      body,
      grid=(pl.cdiv(V, BLKV),),
      in_specs=[
          pl.BlockSpec((E, BLKV), lambda i: (0, i)),
          pl.BlockSpec((E, NP), lambda i: (0, 0)),
      ],
      out_specs=pl.BlockSpec((BLKV * NP // 128, 128), lambda i: (i, 0)),
      out_shape=jax.ShapeDtypeStruct((V * NP // 128, 128), jnp.float32),
  )(table_t, w2)


def _tc_repack_text(text):
  BLK = 2048

  def body(x_ref, o_ref):
    x = x_ref[...]
    xp = jnp.concatenate(
        [x, jnp.zeros((BLK, SP - S), jnp.int32)], axis=1)
    o_ref[...] = xp.reshape(2 * BLK, 128)

  return pl.pallas_call(
      body,
      grid=(B // BLK,),
      in_specs=[pl.BlockSpec((BLK, S), lambda i: (i, 0))],
      out_specs=pl.BlockSpec((2 * BLK, 128), lambda i: (i, 0)),
      out_shape=jax.ShapeDtypeStruct((2 * B, 128), jnp.int32),
  )(text)


def _sc_embed_head(ptab, text2, b16):
  mesh = plsc.VectorSubcoreMesh(
      core_axis_name="c", subcore_axis_name="s", num_cores=NC,
      num_subcores=NS)

  @functools.partial(
      pl.kernel,
      out_type=jax.ShapeDtypeStruct((B, NP), jnp.float32),
      mesh=mesh,
      scratch_types=[
          pltpu.VMEM((2, 2 * CH, 128), jnp.int32),
          pltpu.VMEM((RING, CH), jnp.int32),
          pltpu.VMEM((BPW, NP), jnp.float32),
          pltpu.VMEM((16,), jnp.float32),
          pltpu.SemaphoreType.DMA,
          pltpu.SemaphoreType.DMA,
      ],
      compiler_params=pltpu.CompilerParams(
          use_tc_tiling_on_sc=False, needs_layout_passes=False),
  )
  def body(ptab_hbm, text_hbm, b_hbm, out_hbm, idx_nat, idx_ring, acc_v,
           b_v, sem_idx, sem_g):
    wid = lax.axis_index("s") * NC + lax.axis_index("c")
    base = wid * BPW

    pltpu.async_copy(b_hbm, b_v, sem_idx).wait()

    ar = jnp.arange(16, dtype=jnp.int32)
    zeros = jnp.zeros((16,), jnp.float32)

    def zbody(i, carry):
      acc_v[i, pl.ds(0, 16)] = zeros
      return carry

    lax.fori_loop(0, BPW, zbody, 0, unroll=8)

    def stage(q):
      return pltpu.async_copy(
          text_hbm.at[pl.ds((base + q * CH) * 2, 2 * CH)],
          idx_nat.at[q % 2], sem_idx)

    stage(0).wait()
    for q in range(NQ):
      if q + 1 < NQ:
        stage(q + 1)

      buf = idx_nat.at[q % 2]

      def transpose_step(s, slot, buf=buf):
        row_hi = lax.div(s, 128)
        col = jnp.zeros((16,), jnp.int32) + lax.rem(s, 128)
        for k in range(CH // 16):
          rows = (ar + k * 16) * 2 + row_hi
          idx_ring[slot, pl.ds(k * 16, 16)] = plsc.load_gather(
              buf, [rows, col])

      def fire(slot, q=q):
        pltpu.async_copy(
            ptab_hbm.at[idx_ring.at[slot]],
            acc_v.at[pl.ds(q * CH, CH)],
            sem_g, add=True)

      def drain_one():
        pltpu.make_async_copy(
            ptab_hbm.at[idx_ring.at[0]],
            acc_v.at[pl.ds(0, CH)], sem_g).wait()

      for j in range(NBUF):
        transpose_step(jnp.int32(j), jnp.int32(j))
        fire(jnp.int32(j))

      def gbody(s, carry):
        slot = lax.rem(s, RING)
        transpose_step(s, slot)
        drain_one()
        fire(slot)
        return carry

      lax.fori_loop(NBUF, S, gbody, 0)
      for j in range(NBUF):
        drain_one()

      if q + 1 < NQ:
        pltpu.make_async_copy(
            text_hbm.at[pl.ds(0, 2 * CH)], idx_nat.at[0], sem_idx).wait()

    bvec = b_v[...]

    def ebody(i, carry):
      acc_v[i, pl.ds(0, 16)] = acc_v[i, pl.ds(0, 16)] + bvec
      return carry

    lax.fori_loop(0, BPW, ebody, 0, unroll=8)

    pltpu.async_copy(acc_v, out_hbm.at[pl.ds(base, BPW)], sem_idx).wait()

  return body(ptab, text2, b16)


@jax.jit
def kernel(text, table, W, b):
  w2 = jnp.pad(W * (1.0 / S), ((0, 0), (0, NP - NCLS)))
  b16 = jnp.pad(b, (0, NP - NCLS))
  ptab = _tc_repack_table(table.T, w2)
  text2 = _tc_repack_text(text)
  out16 = _sc_embed_head(ptab.reshape(V, NP), text2, b16)
  return out16[:, :NCLS]


# speedup vs baseline: 2.7355x; 1.8413x over previous
import functools

import jax
import jax.numpy as jnp
from jax import lax
from jax.experimental import pallas as pl
from jax.experimental.pallas import tpu as pltpu
from jax.experimental.pallas import tpu_sc as plsc

B = 16384
S = 200
SP = 256
E = 32
V = 1000000
NCLS = 10
NP = 16

NC = 2
NS = 16
NW = NC * NS
BPW = B // NW
CH = 128
NQ = BPW // CH
NBUF = 14
RING = 16


BLKV = 16384
VPAD = ((V + BLKV - 1) // BLKV) * BLKV


def _tc_repack_table(table_t, w2t):
  def body(x_ref, w_ref, o_ref):
    x = x_ref[...]
    pt = jnp.dot(w_ref[...], x, preferred_element_type=jnp.float32)
    x8 = jnp.concatenate(
        [pt[:, c * (BLKV // 8):(c + 1) * (BLKV // 8)] for c in range(8)],
        axis=0)
    o_ref[...] = jnp.transpose(x8)

  return pl.pallas_call(
      body,
      grid=(pl.cdiv(V, BLKV),),
      in_specs=[
          pl.BlockSpec((E, BLKV), lambda i: (0, i)),
          pl.BlockSpec((NP, E), lambda i: (0, 0)),
      ],
      out_specs=pl.BlockSpec((BLKV // 8, 128), lambda i: (i, 0)),
      out_shape=jax.ShapeDtypeStruct((VPAD * NP // 128, 128), jnp.float32),
  )(table_t, w2t)


def _permute_idx(v):
  vm = v & (BLKV - 1)
  return ((v >> 14) << 14) + ((vm & 2047) << 3) + (vm >> 11)


def _tc_repack_text(text):
  BLK = 2048

  def body(x_ref, o_ref):
    x = _permute_idx(x_ref[...])
    xp = jnp.concatenate(
        [x, jnp.zeros((BLK, SP - S), jnp.int32)], axis=1)
    o_ref[...] = xp.reshape(2 * BLK, 128)

  return pl.pallas_call(
      body,
      grid=(B // BLK,),
      in_specs=[pl.BlockSpec((BLK, S), lambda i: (i, 0))],
      out_specs=pl.BlockSpec((2 * BLK, 128), lambda i: (i, 0)),
      out_shape=jax.ShapeDtypeStruct((2 * B, 128), jnp.int32),
  )(text)


def _sc_embed_head(ptab, text2, b16):
  mesh = plsc.VectorSubcoreMesh(
      core_axis_name="c", subcore_axis_name="s", num_cores=NC,
      num_subcores=NS)

  @functools.partial(
      pl.kernel,
      out_type=jax.ShapeDtypeStruct((B, NP), jnp.float32),
      mesh=mesh,
      scratch_types=[
          pltpu.VMEM((2, 2 * CH, 128), jnp.int32),
          pltpu.VMEM((RING, CH), jnp.int32),
          pltpu.VMEM((BPW, NP), jnp.float32),
          pltpu.VMEM((16,), jnp.float32),
          pltpu.SemaphoreType.DMA,
          pltpu.SemaphoreType.DMA,
      ],
      compiler_params=pltpu.CompilerParams(
          use_tc_tiling_on_sc=False, needs_layout_passes=False),
  )
  def body(ptab_hbm, text_hbm, b_hbm, out_hbm, idx_nat, idx_ring, acc_v,
           b_v, sem_idx, sem_g):
    wid = lax.axis_index("s") * NC + lax.axis_index("c")
    base = wid * BPW

    pltpu.async_copy(b_hbm, b_v, sem_idx).wait()

    ar = jnp.arange(16, dtype=jnp.int32)
    zeros = jnp.zeros((16,), jnp.float32)

    def zbody(i, carry):
      acc_v[i, pl.ds(0, 16)] = zeros
      return carry

    lax.fori_loop(0, BPW, zbody, 0, unroll=8)

    def stage(q):
      return pltpu.async_copy(
          text_hbm.at[pl.ds((base + q * CH) * 2, 2 * CH)],
          idx_nat.at[q % 2], sem_idx)

    stage(0).wait()
    for q in range(NQ):
      if q + 1 < NQ:
        stage(q + 1)

      buf = idx_nat.at[q % 2]

      def transpose_step(s, slot, buf=buf):
        row_hi = lax.div(s, 128)
        col = jnp.zeros((16,), jnp.int32) + lax.rem(s, 128)
        for k in range(CH // 16):
          rows = (ar + k * 16) * 2 + row_hi
          idx_ring[slot, pl.ds(k * 16, 16)] = plsc.load_gather(
              buf, [rows, col])

      def fire(slot, q=q):
        pltpu.async_copy(
            ptab_hbm.at[idx_ring.at[slot]],
            acc_v.at[pl.ds(q * CH, CH)],
            sem_g, add=True)

      def drain_one():
        pltpu.make_async_copy(
            ptab_hbm.at[idx_ring.at[0]],
            acc_v.at[pl.ds(0, CH)], sem_g).wait()

      for j in range(NBUF):
        transpose_step(jnp.int32(j), jnp.int32(j))
        fire(jnp.int32(j))

      def gbody(s, carry):
        slot = lax.rem(s, RING)
        transpose_step(s, slot)
        drain_one()
        fire(slot)
        return carry

      lax.fori_loop(NBUF, S, gbody, 0)
      for j in range(NBUF):
        drain_one()

      if q + 1 < NQ:
        pltpu.make_async_copy(
            text_hbm.at[pl.ds(0, 2 * CH)], idx_nat.at[0], sem_idx).wait()

    bvec = b_v[...]

    def ebody(i, carry):
      acc_v[i, pl.ds(0, 16)] = acc_v[i, pl.ds(0, 16)] + bvec
      return carry

    lax.fori_loop(0, BPW, ebody, 0, unroll=8)

    pltpu.async_copy(acc_v, out_hbm.at[pl.ds(base, BPW)], sem_idx).wait()

  return body(ptab, text2, b16)


@jax.jit
def kernel(text, table, W, b):
  w2t = jnp.pad(W.T * (1.0 / S), ((0, NP - NCLS), (0, 0)))
  b16 = jnp.pad(b, (0, NP - NCLS))
  ptab = _tc_repack_table(table.T, w2t)
  text2 = _tc_repack_text(text)
  out16 = _sc_embed_head(ptab.reshape(VPAD, NP), text2, b16)
  return out16[:, :NCLS]


# speedup vs baseline: 2.9205x; 1.0676x over previous
import functools

import jax
import jax.numpy as jnp
from jax import lax
from jax.experimental import pallas as pl
from jax.experimental.pallas import tpu as pltpu
from jax.experimental.pallas import tpu_sc as plsc

B = 16384
S = 200
E = 32
V = 1000000
NCLS = 10
NP = 16

NC = 2
NS = 16
NW = NC * NS
BPW = B // NW
CH = 128
NQ = BPW // CH
NBUF = 16

BLKV = 16384
VPAD = ((V + BLKV - 1) // BLKV) * BLKV


def _tc_repack_table(table_t, w2t):
  def body(x_ref, w_ref, o_ref):
    x = x_ref[...]
    pt = jnp.dot(w_ref[...], x, preferred_element_type=jnp.float32)
    x8 = jnp.concatenate(
        [pt[:, c * (BLKV // 8):(c + 1) * (BLKV // 8)] for c in range(8)],
        axis=0)
    o_ref[...] = jnp.transpose(x8)

  return pl.pallas_call(
      body,
      grid=(pl.cdiv(V, BLKV),),
      in_specs=[
          pl.BlockSpec((E, BLKV), lambda i: (0, i)),
          pl.BlockSpec((NP, E), lambda i: (0, 0)),
      ],
      out_specs=pl.BlockSpec((BLKV // 8, 128), lambda i: (i, 0)),
      out_shape=jax.ShapeDtypeStruct((VPAD * NP // 128, 128), jnp.float32),
  )(table_t, w2t)


def _permute_idx(v):
  vm = v & (BLKV - 1)
  return ((v >> 14) << 14) + ((vm & 2047) << 3) + (vm >> 11)


def _tc_permute_text(text_t):
  BLK = 4096

  def body(x_ref, o_ref):
    o_ref[...] = _permute_idx(x_ref[...])

  return pl.pallas_call(
      body,
      grid=(B // BLK,),
      in_specs=[pl.BlockSpec((S, BLK), lambda i: (0, i))],
      out_specs=pl.BlockSpec((S, BLK), lambda i: (0, i)),
      out_shape=jax.ShapeDtypeStruct((S, B), jnp.int32),
  )(text_t)


def _sc_embed_head(ptab, textp, b16):
  mesh = plsc.VectorSubcoreMesh(
      core_axis_name="c", subcore_axis_name="s", num_cores=NC,
      num_subcores=NS)

  @functools.partial(
      pl.kernel,
      out_type=jax.ShapeDtypeStruct((B, NP), jnp.float32),
      mesh=mesh,
      scratch_types=[
          pltpu.VMEM((2, S, CH), jnp.int32),
          pltpu.VMEM((BPW, NP), jnp.float32),
          pltpu.VMEM((16,), jnp.float32),
          pltpu.SemaphoreType.DMA,
          pltpu.SemaphoreType.DMA,
      ],
      compiler_params=pltpu.CompilerParams(
          use_tc_tiling_on_sc=False, needs_layout_passes=False),
  )
  def body(ptab_hbm, text_hbm, b_hbm, out_hbm, idx_v, acc_v, b_v,
           sem_idx, sem_g):
    wid = lax.axis_index("s") * NC + lax.axis_index("c")
    base = wid * BPW

    pltpu.async_copy(b_hbm, b_v, sem_idx).wait()

    zeros = jnp.zeros((16,), jnp.float32)

    def zbody(i, carry):
      acc_v[i, pl.ds(0, 16)] = zeros
      return carry

    lax.fori_loop(0, BPW, zbody, 0, unroll=8)

    def stage(q):
      return pltpu.async_copy(
          text_hbm.at[:, pl.ds(base + q * CH, CH)],
          idx_v.at[q % 2], sem_idx)

    stage(0).wait()
    for q in range(NQ):
      if q + 1 < NQ:
        stage(q + 1)

      buf = idx_v.at[q % 2]

      def fire(s, q=q, buf=buf):
        pltpu.async_copy(
            ptab_hbm.at[buf.at[s]],
            acc_v.at[pl.ds(q * CH, CH)],
            sem_g, add=True)

      def drain_one():
        pltpu.make_async_copy(
            ptab_hbm.at[idx_v.at[0, 0]],
            acc_v.at[pl.ds(0, CH)], sem_g).wait()

      for j in range(NBUF):
        fire(jnp.int32(j))

      def gbody(s, carry):
        drain_one()
        fire(s)
        return carry

      lax.fori_loop(NBUF, S, gbody, 0)
      for j in range(NBUF):
        drain_one()

      if q + 1 < NQ:
        pltpu.make_async_copy(
            text_hbm.at[:, pl.ds(0, CH)], idx_v.at[0], sem_idx).wait()

    bvec = b_v[...]

    def ebody(i, carry):
      acc_v[i, pl.ds(0, 16)] = acc_v[i, pl.ds(0, 16)] + bvec
      return carry

    lax.fori_loop(0, BPW, ebody, 0, unroll=8)

    pltpu.async_copy(acc_v, out_hbm.at[pl.ds(base, BPW)], sem_idx).wait()

  return body(ptab, textp, b16)


@jax.jit
def kernel(text, table, W, b):
  w2t = jnp.pad(W.T * (1.0 / S), ((0, NP - NCLS), (0, 0)))
  b16 = jnp.pad(b, (0, NP - NCLS))
  ptab = _tc_repack_table(table.T, w2t)
  textp = _tc_permute_text(text.T)
  out16 = _sc_embed_head(ptab.reshape(VPAD, NP), textp, b16)
  return out16[:, :NCLS]


# speedup vs baseline: 2.9563x; 1.0123x over previous
import functools

import jax
import jax.numpy as jnp
from jax import lax
from jax.experimental import pallas as pl
from jax.experimental.pallas import tpu as pltpu
from jax.experimental.pallas import tpu_sc as plsc

B = 16384
S = 200
E = 32
V = 1000000
NCLS = 10
NP = 16

NC = 2
NS = 16
NW = NC * NS
BPW = B // NW
CH = 128
NQ = BPW // CH
NBUF = 16

BLKV = 16384
VPAD = ((V + BLKV - 1) // BLKV) * BLKV


def _tc_repack_table(table_t, w2t):
  def body(x_ref, w_ref, o_ref):
    x = x_ref[...]
    pt = jnp.dot(w_ref[...], x, preferred_element_type=jnp.float32)
    x8 = jnp.concatenate(
        [pt[:, c * (BLKV // 8):(c + 1) * (BLKV // 8)] for c in range(8)],
        axis=0)
    o_ref[...] = jnp.transpose(x8)

  return pl.pallas_call(
      body,
      grid=(pl.cdiv(V, BLKV),),
      in_specs=[
          pl.BlockSpec((E, BLKV), lambda i: (0, i)),
          pl.BlockSpec((NP, E), lambda i: (0, 0)),
      ],
      out_specs=pl.BlockSpec((BLKV // 8, 128), lambda i: (i, 0)),
      out_shape=jax.ShapeDtypeStruct((VPAD * NP // 128, 128), jnp.float32),
  )(table_t, w2t)


def _permute_idx(v):
  vm = v & (BLKV - 1)
  return ((v >> 14) << 14) + ((vm & 2047) << 3) + (vm >> 11)


def _tc_permute_text(text_t):
  BLKS = 40

  def body(x_ref, o_ref):
    x = _permute_idx(x_ref[...])
    o_ref[...] = x.reshape(BLKS * B // 128, 128)

  return pl.pallas_call(
      body,
      grid=(S // BLKS,),
      in_specs=[pl.BlockSpec((BLKS, B), lambda i: (i, 0))],
      out_specs=pl.BlockSpec((BLKS * B // 128, 128), lambda i: (i, 0)),
      out_shape=jax.ShapeDtypeStruct((S * B // 128, 128), jnp.int32),
  )(text_t)


def _sc_embed_head(ptab, textp, b16):
  mesh = plsc.VectorSubcoreMesh(
      core_axis_name="c", subcore_axis_name="s", num_cores=NC,
      num_subcores=NS)

  @functools.partial(
      pl.kernel,
      out_type=jax.ShapeDtypeStruct((B, NP), jnp.float32),
      mesh=mesh,
      scratch_types=[
          pltpu.VMEM((2, S, 128), jnp.int32),
          pltpu.VMEM((BPW, NP), jnp.float32),
          pltpu.VMEM((16,), jnp.float32),
          pltpu.SemaphoreType.DMA,
          pltpu.SemaphoreType.DMA,
      ],
      compiler_params=pltpu.CompilerParams(
          use_tc_tiling_on_sc=False, needs_layout_passes=False),
  )
  def body(ptab_hbm, text_hbm, b_hbm, out_hbm, idx_v, acc_v, b_v,
           sem_idx, sem_g):
    wid = lax.axis_index("s") * NC + lax.axis_index("c")
    base = wid * BPW

    pltpu.async_copy(b_hbm, b_v, sem_idx).wait()

    zeros = jnp.zeros((16,), jnp.float32)

    def zbody(i, carry):
      acc_v[i, pl.ds(0, 16)] = zeros
      return carry

    lax.fori_loop(0, BPW, zbody, 0, unroll=8)

    def stage(q):
      return pltpu.async_copy(
          text_hbm.at[:, wid * NQ + q, :],
          idx_v.at[q % 2], sem_idx)

    stage(0).wait()
    for q in range(NQ):
      if q + 1 < NQ:
        stage(q + 1)

      buf = idx_v.at[q % 2]

      def fire(s, q=q, buf=buf):
        pltpu.async_copy(
            ptab_hbm.at[buf.at[s]],
            acc_v.at[pl.ds(q * CH, CH)],
            sem_g, add=True)

      def drain_one():
        pltpu.make_async_copy(
            ptab_hbm.at[idx_v.at[0, 0]],
            acc_v.at[pl.ds(0, CH)], sem_g).wait()

      for j in range(NBUF):
        fire(jnp.int32(j))

      def gbody(s, carry):
        drain_one()
        fire(s)
        return carry

      lax.fori_loop(NBUF, S, gbody, 0)
      for j in range(NBUF):
        drain_one()

      if q + 1 < NQ:
        pltpu.make_async_copy(
            text_hbm.at[:, 0, :], idx_v.at[0], sem_idx).wait()

    bvec = b_v[...]

    def ebody(i, carry):
      acc_v[i, pl.ds(0, 16)] = acc_v[i, pl.ds(0, 16)] + bvec
      return carry

    lax.fori_loop(0, BPW, ebody, 0, unroll=8)

    pltpu.async_copy(acc_v, out_hbm.at[pl.ds(base, BPW)], sem_idx).wait()

  return body(ptab, textp, b16)


@jax.jit
def kernel(text, table, W, b):
  w2t = jnp.pad(W.T * (1.0 / S), ((0, NP - NCLS), (0, 0)))
  b16 = jnp.pad(b, (0, NP - NCLS))
  ptab = _tc_repack_table(table.T, w2t)
  textp = _tc_permute_text(text.T)
  out16 = _sc_embed_head(
      ptab.reshape(VPAD, NP), textp.reshape(S, B // 128, 128), b16)
  return out16[:, :NCLS]
